# baseline (device time: 242368 ns/iter reference)
import jax
import jax.numpy as jnp
from jax import lax
from jax.experimental import pallas as pl
from jax.experimental.pallas import tpu as pltpu

N_DEV = 8
N_TOK = 2048
D = 1024
E_LOCAL = 8
N_EXP = 64
CHUNK = N_TOK // N_DEV
N_STEPS = 2 * (N_DEV - 1)


def _compute_body(x_ref, rw_ref, idx_ref, ew_ref, sw_ref, out_ref, gate_ref):
    e = pl.program_id(0)
    d = lax.axis_index("i")

    @pl.when(e == 0)
    def _():
        scores = jnp.dot(
            x_ref[:, :], rw_ref[:, :], preferred_element_type=jnp.float32
        )
        m = jnp.max(scores, axis=-1, keepdims=True)
        p = jnp.exp(scores - m)
        probs = p / jnp.sum(p, axis=-1, keepdims=True)
        sel = (
            lax.broadcasted_iota(jnp.int32, (N_TOK, N_EXP), 1) == idx_ref[:, :]
        )
        gate_ref[:, :] = jnp.sum(
            jnp.where(sel, probs, 0.0), axis=-1, keepdims=True
        )
        out_ref[:, :] = jnp.zeros_like(out_ref)
        row0 = d * CHUNK
        out_ref[pl.ds(row0, CHUNK), :] = jnp.dot(
            x_ref[pl.ds(row0, CHUNK), :],
            sw_ref[:, :],
            preferred_element_type=jnp.float32,
        )

    g = d * E_LOCAL + e
    w = jnp.where(idx_ref[:, :] == g, gate_ref[:, :], 0.0)
    y = jnp.dot(x_ref[:, :], ew_ref[0, :, :], preferred_element_type=jnp.float32)
    out_ref[:, :] += w * y


def _allreduce_body(p_ref, out_ref, comm_ref, send_sems, recv_sems):
    d = lax.axis_index("i")
    left = lax.rem(d + N_DEV - 1, N_DEV)
    right = lax.rem(d + 1, N_DEV)

    barrier = pltpu.get_barrier_semaphore()
    for nbr in (left, right):
        pl.semaphore_signal(
            barrier, inc=1, device_id=(nbr,), device_id_type=pl.DeviceIdType.MESH
        )
    pl.semaphore_wait(barrier, 2)

    comm_ref[0, :, :] = p_ref[pl.ds(d * CHUNK, CHUNK), :]
    for k in range(N_STEPS):
        if 1 <= k <= N_DEV - 1:
            c = lax.rem(d - k + 2 * N_DEV, N_DEV)
            comm_ref[k, :, :] += p_ref[pl.ds(c * CHUNK, CHUNK), :]
        if k == N_DEV - 1:
            c = lax.rem(d + 1, N_DEV)
            out_ref[pl.ds(c * CHUNK, CHUNK), :] = comm_ref[k, :, :]
        rdma = pltpu.make_async_remote_copy(
            src_ref=comm_ref.at[k],
            dst_ref=comm_ref.at[k + 1],
            send_sem=send_sems.at[k],
            recv_sem=recv_sems.at[k],
            device_id=(right,),
            device_id_type=pl.DeviceIdType.MESH,
        )
        rdma.start()
        rdma.wait()
        if k >= N_DEV - 1:
            c = lax.rem(d - (k - (N_DEV - 1)) + 2 * N_DEV, N_DEV)
            out_ref[pl.ds(c * CHUNK, CHUNK), :] = comm_ref[k + 1, :, :]


def kernel(x, router_W, route_idx, expert_W, shared_W):
    partial = pl.pallas_call(
        _compute_body,
        grid=(E_LOCAL,),
        in_specs=[
            pl.BlockSpec((N_TOK, D), lambda e: (0, 0)),
            pl.BlockSpec((D, N_EXP), lambda e: (0, 0)),
            pl.BlockSpec((N_TOK, 1), lambda e: (0, 0)),
            pl.BlockSpec((1, D, D), lambda e: (e, 0, 0)),
            pl.BlockSpec((D, D), lambda e: (0, 0)),
        ],
        out_specs=pl.BlockSpec((N_TOK, D), lambda e: (0, 0)),
        out_shape=jax.ShapeDtypeStruct((N_TOK, D), jnp.float32),
        scratch_shapes=[pltpu.VMEM((N_TOK, 1), jnp.float32)],
        compiler_params=pltpu.CompilerParams(
            dimension_semantics=("arbitrary",),
        ),
    )(x, router_W, route_idx, expert_W, shared_W)

    return pl.pallas_call(
        _allreduce_body,
        out_shape=jax.ShapeDtypeStruct((N_TOK, D), jnp.float32),
        in_specs=[pl.BlockSpec(memory_space=pltpu.VMEM)],
        out_specs=pl.BlockSpec(memory_space=pltpu.VMEM),
        scratch_shapes=[
            pltpu.VMEM((N_STEPS + 1, CHUNK, D), jnp.float32),
            pltpu.SemaphoreType.DMA((N_STEPS,)),
            pltpu.SemaphoreType.DMA((N_STEPS,)),
        ],
        compiler_params=pltpu.CompilerParams(collective_id=0),
    )(partial)


# device time: 193699 ns/iter; 1.2513x vs baseline; 1.2513x over previous
import jax
import jax.numpy as jnp
from jax import lax
from jax.experimental import pallas as pl
from jax.experimental.pallas import tpu as pltpu

N_DEV = 8
N_TOK = 2048
D = 1024
E_LOCAL = 8
N_EXP = 64
CHUNK = N_TOK // N_DEV
N_STEPS = 2 * (N_DEV - 1)


def _compute_body(x_ref, rw_ref, idx_ref, ew_ref, sw_ref, out_ref, gate_ref):
    e = pl.program_id(0)
    d = lax.axis_index("i")

    @pl.when(e == 0)
    def _():
        scores = jnp.dot(
            x_ref[:, :], rw_ref[:, :], preferred_element_type=jnp.float32
        )
        m = jnp.max(scores, axis=-1, keepdims=True)
        p = jnp.exp(scores - m)
        probs = p / jnp.sum(p, axis=-1, keepdims=True)
        sel = (
            lax.broadcasted_iota(jnp.int32, (N_TOK, N_EXP), 1) == idx_ref[:, :]
        )
        gate_ref[:, :] = jnp.sum(
            jnp.where(sel, probs, 0.0), axis=-1, keepdims=True
        )
        out_ref[:, :] = jnp.zeros_like(out_ref)
        row0 = d * CHUNK
        out_ref[pl.ds(row0, CHUNK), :] = jnp.dot(
            x_ref[pl.ds(row0, CHUNK), :],
            sw_ref[:, :],
            preferred_element_type=jnp.float32,
        )

    g = d * E_LOCAL + e
    w = jnp.where(idx_ref[:, :] == g, gate_ref[:, :], 0.0)
    y = jnp.dot(x_ref[:, :], ew_ref[0, :, :], preferred_element_type=jnp.float32)
    out_ref[:, :] += w * y


def _allreduce_body(p_ref, out_ref, comm_ref, send_sems, recv_sems):
    d = lax.axis_index("i")
    left = lax.rem(d + N_DEV - 1, N_DEV)
    right = lax.rem(d + 1, N_DEV)

    barrier = pltpu.get_barrier_semaphore()
    for nbr in (left, right):
        pl.semaphore_signal(
            barrier, inc=1, device_id=(nbr,), device_id_type=pl.DeviceIdType.MESH
        )
    pl.semaphore_wait(barrier, 2)

    comm_ref[0, :, :] = p_ref[pl.ds(d * CHUNK, CHUNK), :]
    for k in range(N_STEPS):
        if 1 <= k <= N_DEV - 1:
            c = lax.rem(d - k + 2 * N_DEV, N_DEV)
            comm_ref[k, :, :] += p_ref[pl.ds(c * CHUNK, CHUNK), :]
        if k == N_DEV - 1:
            c = lax.rem(d + 1, N_DEV)
            out_ref[pl.ds(c * CHUNK, CHUNK), :] = comm_ref[k, :, :]
        rdma = pltpu.make_async_remote_copy(
            src_ref=comm_ref.at[k],
            dst_ref=comm_ref.at[k + 1],
            send_sem=send_sems.at[k],
            recv_sem=recv_sems.at[k],
            device_id=(right,),
            device_id_type=pl.DeviceIdType.MESH,
        )
        rdma.start()
        rdma.wait()
        if k >= N_DEV - 1:
            c = lax.rem(d - (k - (N_DEV - 1)) + 2 * N_DEV, N_DEV)
            out_ref[pl.ds(c * CHUNK, CHUNK), :] = comm_ref[k + 1, :, :]


import os
_AR_ONLY = os.environ.get("SCBAND_AR_ONLY") == "1"


def kernel(x, router_W, route_idx, expert_W, shared_W):
    if _AR_ONLY:
        return _allreduce(x)
    partial = pl.pallas_call(
        _compute_body,
        grid=(E_LOCAL,),
        in_specs=[
            pl.BlockSpec((N_TOK, D), lambda e: (0, 0)),
            pl.BlockSpec((D, N_EXP), lambda e: (0, 0)),
            pl.BlockSpec((N_TOK, 1), lambda e: (0, 0)),
            pl.BlockSpec((1, D, D), lambda e: (e, 0, 0)),
            pl.BlockSpec((D, D), lambda e: (0, 0)),
        ],
        out_specs=pl.BlockSpec((N_TOK, D), lambda e: (0, 0)),
        out_shape=jax.ShapeDtypeStruct((N_TOK, D), jnp.float32),
        scratch_shapes=[pltpu.VMEM((N_TOK, 1), jnp.float32)],
        compiler_params=pltpu.CompilerParams(
            dimension_semantics=("arbitrary",),
        ),
    )(x, router_W, route_idx, expert_W, shared_W)

    return _allreduce(partial)


def _allreduce(partial):
    return pl.pallas_call(
        _allreduce_body,
        out_shape=jax.ShapeDtypeStruct((N_TOK, D), jnp.float32),
        in_specs=[pl.BlockSpec(memory_space=pltpu.VMEM)],
        out_specs=pl.BlockSpec(memory_space=pltpu.VMEM),
        scratch_shapes=[
            pltpu.VMEM((N_STEPS + 1, CHUNK, D), jnp.float32),
            pltpu.SemaphoreType.DMA((N_STEPS,)),
            pltpu.SemaphoreType.DMA((N_STEPS,)),
        ],
        compiler_params=pltpu.CompilerParams(collective_id=0),
    )(partial)


# device time: 130088 ns/iter; 1.8631x vs baseline; 1.4890x over previous
import jax
import jax.numpy as jnp
from jax import lax
from jax.experimental import pallas as pl
from jax.experimental.pallas import tpu as pltpu

N_DEV = 8
N_TOK = 2048
D = 1024
E_LOCAL = 8
N_EXP = 64
CHUNK = N_TOK // N_DEV
N_STEPS = 2 * (N_DEV - 1)


def _compute_body(x_ref, rw_ref, idx_ref, ew_ref, sw_ref, out_ref, gate_ref):
    e = pl.program_id(0)
    d = lax.axis_index("i")

    @pl.when(e == 0)
    def _():
        scores = jnp.dot(
            x_ref[:, :], rw_ref[:, :], preferred_element_type=jnp.float32
        )
        m = jnp.max(scores, axis=-1, keepdims=True)
        p = jnp.exp(scores - m)
        probs = p / jnp.sum(p, axis=-1, keepdims=True)
        sel = (
            lax.broadcasted_iota(jnp.int32, (N_TOK, N_EXP), 1) == idx_ref[:, :]
        )
        gate_ref[:, :] = jnp.sum(
            jnp.where(sel, probs, 0.0), axis=-1, keepdims=True
        )
        out_ref[:, :] = jnp.zeros_like(out_ref)
        row0 = d * CHUNK
        out_ref[pl.ds(row0, CHUNK), :] = jnp.dot(
            x_ref[pl.ds(row0, CHUNK), :],
            sw_ref[:, :],
            preferred_element_type=jnp.float32,
        )

    g = d * E_LOCAL + e
    w = jnp.where(idx_ref[:, :] == g, gate_ref[:, :], 0.0)
    y = jnp.dot(x_ref[:, :], ew_ref[0, :, :], preferred_element_type=jnp.float32)
    out_ref[:, :] += w * y


_PARTS = (
    (0, 96, (4, 2, 1)),
    (768, 96, (2, 1, 4)),
    (1536, 64, (1, 4, 2)),
)
_RS_SLOT_BASE = (0, 4, 6)
_N_SEMS = 42


def _subset_sums(masks):
    out = [0]
    for m in masks:
        out = out + [s + m for s in out]
    return out


def _allreduce_body(p_ref, out_ref, rs_ref, send_sems, recv_sems):
    d = lax.axis_index("i")
    bx = (d ^ (d >> 1)) & 1
    by = (d >> 1) & 1
    bz = (d >> 2) & 1
    n = bx + 2 * by + 4 * bz

    def pos_of(nn):
        px = nn & 1
        py = (nn >> 1) & 1
        pz = (nn >> 2) & 1
        return 4 * pz + 2 * py + (px ^ py)

    partner_pos = {m: pos_of(n ^ m) for m in (1, 2, 4)}

    out_ref[:, :] = p_ref[:, :]

    barrier = pltpu.get_barrier_semaphore()
    for m in (1, 2, 4):
        pl.semaphore_signal(
            barrier,
            inc=1,
            device_id=(partner_pos[m],),
            device_id_type=pl.DeviceIdType.MESH,
        )
    pl.semaphore_wait(barrier, 3)

    sem_ctr = [0]

    def next_sem():
        i = sem_ctr[0]
        sem_ctr[0] += 1
        return i

    for j in range(3):
        stage = []
        for k, (base, sub, masks) in enumerate(_PARTS):
            mj = masks[j]
            cm = sum(masks[:j])
            send_cons = (n & cm) | ((n & mj) ^ mj)
            recv_cons = n & (cm | mj)
            for idx, f in enumerate(_subset_sums(masks[j + 1 :])):
                si = next_sem()
                slot = _RS_SLOT_BASE[j] + idx
                rdma = pltpu.make_async_remote_copy(
                    src_ref=out_ref.at[pl.ds((base // sub + send_cons + f) * sub, sub), :],
                    dst_ref=rs_ref.at[k, pl.ds(slot * sub, sub), :],
                    send_sem=send_sems.at[si],
                    recv_sem=recv_sems.at[si],
                    device_id=(partner_pos[mj],),
                    device_id_type=pl.DeviceIdType.MESH,
                )
                rdma.start()
                stage.append((rdma, k, base, sub, recv_cons + f, slot))
        for rdma, k, base, sub, v, slot in stage:
            rdma.wait()
            out_ref[pl.ds(base + v * sub, sub), :] += rs_ref[
                k, pl.ds(slot * sub, sub), :
            ]

    for r in range(3):
        stage = []
        for base, sub, masks in _PARTS:
            for f in _subset_sums(masks[3 - r :]):
                si = next_sem()
                row = base + (jnp.bitwise_xor(n, f)) * sub
                rdma = pltpu.make_async_remote_copy(
                    src_ref=out_ref.at[pl.ds(row, sub), :],
                    dst_ref=out_ref.at[pl.ds(row, sub), :],
                    send_sem=send_sems.at[si],
                    recv_sem=recv_sems.at[si],
                    device_id=(partner_pos[masks[2 - r]],),
                    device_id_type=pl.DeviceIdType.MESH,
                )
                rdma.start()
                stage.append(rdma)
        for rdma in stage:
            rdma.wait()


def _allreduce_body_ring(p_ref, out_ref, comm_ref, send_sems, recv_sems):
    d = lax.axis_index("i")
    left = lax.rem(d + N_DEV - 1, N_DEV)
    right = lax.rem(d + 1, N_DEV)

    barrier = pltpu.get_barrier_semaphore()
    for nbr in (left, right):
        pl.semaphore_signal(
            barrier, inc=1, device_id=(nbr,), device_id_type=pl.DeviceIdType.MESH
        )
    pl.semaphore_wait(barrier, 2)

    comm_ref[0, :, :] = p_ref[pl.ds(d * CHUNK, CHUNK), :]
    for k in range(N_STEPS):
        if 1 <= k <= N_DEV - 1:
            c = lax.rem(d - k + 2 * N_DEV, N_DEV)
            comm_ref[k, :, :] += p_ref[pl.ds(c * CHUNK, CHUNK), :]
        if k == N_DEV - 1:
            c = lax.rem(d + 1, N_DEV)
            out_ref[pl.ds(c * CHUNK, CHUNK), :] = comm_ref[k, :, :]
        rdma = pltpu.make_async_remote_copy(
            src_ref=comm_ref.at[k],
            dst_ref=comm_ref.at[k + 1],
            send_sem=send_sems.at[k],
            recv_sem=recv_sems.at[k],
            device_id=(right,),
            device_id_type=pl.DeviceIdType.MESH,
        )
        rdma.start()
        rdma.wait()
        if k >= N_DEV - 1:
            c = lax.rem(d - (k - (N_DEV - 1)) + 2 * N_DEV, N_DEV)
            out_ref[pl.ds(c * CHUNK, CHUNK), :] = comm_ref[k + 1, :, :]


import os
_AR_ONLY = os.environ.get("SCBAND_AR_ONLY") == "1"


def kernel(x, router_W, route_idx, expert_W, shared_W):
    if _AR_ONLY:
        return _allreduce(x)
    partial = pl.pallas_call(
        _compute_body,
        grid=(E_LOCAL,),
        in_specs=[
            pl.BlockSpec((N_TOK, D), lambda e: (0, 0)),
            pl.BlockSpec((D, N_EXP), lambda e: (0, 0)),
            pl.BlockSpec((N_TOK, 1), lambda e: (0, 0)),
            pl.BlockSpec((1, D, D), lambda e: (e, 0, 0)),
            pl.BlockSpec((D, D), lambda e: (0, 0)),
        ],
        out_specs=pl.BlockSpec((N_TOK, D), lambda e: (0, 0)),
        out_shape=jax.ShapeDtypeStruct((N_TOK, D), jnp.float32),
        scratch_shapes=[pltpu.VMEM((N_TOK, 1), jnp.float32)],
        compiler_params=pltpu.CompilerParams(
            dimension_semantics=("arbitrary",),
        ),
    )(x, router_W, route_idx, expert_W, shared_W)

    return _allreduce(partial)


def _allreduce(partial):
    return pl.pallas_call(
        _allreduce_body,
        out_shape=jax.ShapeDtypeStruct((N_TOK, D), jnp.float32),
        in_specs=[pl.BlockSpec(memory_space=pltpu.VMEM)],
        out_specs=pl.BlockSpec(memory_space=pltpu.VMEM),
        scratch_shapes=[
            pltpu.VMEM((3, 7 * 96, D), jnp.float32),
            pltpu.SemaphoreType.DMA((_N_SEMS,)),
            pltpu.SemaphoreType.DMA((_N_SEMS,)),
        ],
        compiler_params=pltpu.CompilerParams(collective_id=0),
    )(partial)
